# baseline (speedup 1.0000x reference)
import functools

import jax
import jax.numpy as jnp
from jax import lax
from jax.experimental import pallas as pl
from jax.experimental.pallas import tpu as pltpu
from jax.experimental.pallas import tpu_sc as plsc

BATCH = 16384
EMBED = 32
HIDDEN = 64
LINE = 128
RPL = LINE // EMBED

_info = plsc.get_sparse_core_info()
_NC, _NS = _info.num_cores, _info.num_subcores
_NW = _NC * _NS
_BPW = BATCH // _NW
_L = _info.num_lanes

_mesh = plsc.VectorSubcoreMesh(core_axis_name="c", subcore_axis_name="s")


@functools.partial(
    pl.kernel,
    mesh=_mesh,
    out_type=(
        jax.ShapeDtypeStruct((BATCH, LINE), jnp.float32),
        jax.ShapeDtypeStruct((BATCH, LINE), jnp.float32),
    ),
    scratch_types=[
        pltpu.VMEM((_BPW,), jnp.int32),
        pltpu.VMEM((_BPW,), jnp.int32),
        pltpu.VMEM((_BPW,), jnp.int32),
        pltpu.VMEM((_BPW,), jnp.int32),
        pltpu.VMEM((_BPW, LINE), jnp.float32),
        pltpu.SemaphoreType.DMA,
    ],
)
def _sc_gather(uid_hbm, pid_hbm, ut_hbm, pt_hbm, uo_hbm, po_hbm,
               uidx_v, pidx_v, ulin_v, plin_v, lines_v, sem):
    wid = lax.axis_index("s") * _NC + lax.axis_index("c")
    base = wid * _BPW
    pltpu.sync_copy(uid_hbm.at[pl.ds(base, _BPW)], uidx_v)
    pltpu.sync_copy(pid_hbm.at[pl.ds(base, _BPW)], pidx_v)

    def _shift(g, carry):
        s = pl.ds(g * _L, _L)
        ulin_v[s] = lax.shift_right_logical(uidx_v[s], 2)
        plin_v[s] = lax.shift_right_logical(pidx_v[s], 2)
        return carry

    lax.fori_loop(0, _BPW // _L, _shift, 0)

    cu = pltpu.async_copy(ut_hbm.at[ulin_v], lines_v, sem)
    cu.wait()
    pltpu.sync_copy(lines_v, uo_hbm.at[pl.ds(base, _BPW)])
    cp = pltpu.async_copy(pt_hbm.at[plin_v], lines_v, sem)
    cp.wait()
    pltpu.sync_copy(lines_v, po_hbm.at[pl.ds(base, _BPW)])


_BLK = 2048


def _mlp_body(ids_ref, ul_ref, pl_ref, w1u_ref, w1p_ref, b1_ref, w2_ref,
              b2_ref, o_ref):
    uoff = ids_ref[:, 0:1] & (RPL - 1)
    poff = ids_ref[:, 1:2] & (RPL - 1)
    lu = ul_ref[...]
    lp = pl_ref[...]
    uemb = (uoff == 0).astype(jnp.float32) * lu[:, 0:EMBED]
    pemb = (poff == 0).astype(jnp.float32) * lp[:, 0:EMBED]
    for k in range(1, RPL):
        uemb += (uoff == k).astype(jnp.float32) * lu[:, k * EMBED:(k + 1) * EMBED]
        pemb += (poff == k).astype(jnp.float32) * lp[:, k * EMBED:(k + 1) * EMBED]
    x = (jnp.dot(uemb, w1u_ref[...], preferred_element_type=jnp.float32)
         + jnp.dot(pemb, w1p_ref[...], preferred_element_type=jnp.float32)
         + b1_ref[...])
    x = jnp.maximum(x, 0.0)
    o_ref[...] = jnp.sum(x * w2_ref[...], axis=1, keepdims=True) + b2_ref[...]


def _mlp(ids2, u_lines, p_lines, w1u, w1p, b1_2d, w2t, b2_2d):
    grid = (BATCH // _BLK,)
    return pl.pallas_call(
        _mlp_body,
        grid=grid,
        in_specs=[
            pl.BlockSpec((_BLK, 2), lambda i: (i, 0)),
            pl.BlockSpec((_BLK, LINE), lambda i: (i, 0)),
            pl.BlockSpec((_BLK, LINE), lambda i: (i, 0)),
            pl.BlockSpec((EMBED, HIDDEN), lambda i: (0, 0)),
            pl.BlockSpec((EMBED, HIDDEN), lambda i: (0, 0)),
            pl.BlockSpec((1, HIDDEN), lambda i: (0, 0)),
            pl.BlockSpec((1, HIDDEN), lambda i: (0, 0)),
            pl.BlockSpec((1, 1), lambda i: (0, 0)),
        ],
        out_specs=pl.BlockSpec((_BLK, 1), lambda i: (i, 0)),
        out_shape=jax.ShapeDtypeStruct((BATCH, 1), jnp.float32),
    )(ids2, u_lines, p_lines, w1u, w1p, b1_2d, w2t, b2_2d)


def kernel(user_ids, post_ids, user_table, post_table, W1, b1, W2, b2):
    ut2 = user_table.reshape(-1, LINE)
    pt2 = post_table.reshape(-1, LINE)
    u_lines, p_lines = _sc_gather(user_ids, post_ids, ut2, pt2)
    ids2 = jnp.stack([user_ids, post_ids], axis=1)
    return _mlp(
        ids2, u_lines, p_lines,
        W1[:EMBED], W1[EMBED:],
        b1.reshape(1, HIDDEN),
        W2.reshape(1, HIDDEN),
        b2.reshape(1, 1),
    )


# speedup vs baseline: 1.4579x; 1.4579x over previous
import functools

import jax
import jax.numpy as jnp
from jax import lax
from jax.experimental import pallas as pl
from jax.experimental.pallas import tpu as pltpu
from jax.experimental.pallas import tpu_sc as plsc

BATCH = 16384
EMBED = 32
HIDDEN = 64
LINE = 128
RPL = LINE // EMBED
NROWS = 1000000
NLINES = NROWS // RPL

_info = plsc.get_sparse_core_info()
_NC, _NS = _info.num_cores, _info.num_subcores
_NW = _NC * _NS
_BPW = BATCH // _NW
_L = _info.num_lanes

_mesh = plsc.VectorSubcoreMesh(core_axis_name="c", subcore_axis_name="s")

_TCH = 4096


_NSB = (NROWS + _TCH - 1) // _TCH
_QPB = _TCH // RPL
_NLINES_P = _NSB * _QPB


def _pack_body(t_ref, o_ref):
    x = t_ref[...]
    parts = [x[:, a * _QPB:(a + 1) * _QPB].T for a in range(RPL)]
    o_ref[...] = jnp.concatenate(parts, axis=1)


def _pack(tT):
    return pl.pallas_call(
        _pack_body,
        grid=(_NSB,),
        in_specs=[pl.BlockSpec((EMBED, _TCH), lambda i: (0, i))],
        out_specs=pl.BlockSpec((_QPB, LINE), lambda i: (i, 0)),
        out_shape=jax.ShapeDtypeStruct((_NLINES_P, LINE), jnp.float32),
    )(tT)


@functools.partial(
    pl.kernel,
    mesh=_mesh,
    out_type=jax.ShapeDtypeStruct((BATCH, LINE), jnp.float32),
    scratch_types=[
        pltpu.VMEM((_BPW,), jnp.int32),
        pltpu.VMEM((_BPW,), jnp.int32),
        pltpu.VMEM((_BPW, LINE), jnp.float32),
        pltpu.SemaphoreType.DMA,
    ],
)
def _sc_gather(id_hbm, tbl_hbm, o_hbm, idx_v, lin_v, lines_v, sem):
    wid = lax.axis_index("s") * _NC + lax.axis_index("c")
    base = wid * _BPW
    pltpu.sync_copy(id_hbm.at[pl.ds(base, _BPW)], idx_v)

    def _shift(g, carry):
        s = pl.ds(g * _L, _L)
        v = idx_v[s]
        lin_v[s] = lax.shift_left(lax.shift_right_logical(v, 12), 10) | (v & (_QPB - 1))
        return carry

    lax.fori_loop(0, _BPW // _L, _shift, 0)

    pltpu.async_copy(tbl_hbm.at[lin_v], lines_v, sem).wait()
    pltpu.sync_copy(lines_v, o_hbm.at[pl.ds(base, _BPW)])


_BLK = 2048


def _mlp_body(ids_ref, ul_ref, pl_ref, w1u_ref, w1p_ref, b1_ref, w2_ref,
              b2_ref, o_ref):
    uoff = lax.shift_right_logical(ids_ref[:, 0:1], 10) & (RPL - 1)
    poff = lax.shift_right_logical(ids_ref[:, 1:2], 10) & (RPL - 1)
    lu = ul_ref[...]
    lp = pl_ref[...]
    uemb = (uoff == 0).astype(jnp.float32) * lu[:, 0:EMBED]
    pemb = (poff == 0).astype(jnp.float32) * lp[:, 0:EMBED]
    for k in range(1, RPL):
        uemb += (uoff == k).astype(jnp.float32) * lu[:, k * EMBED:(k + 1) * EMBED]
        pemb += (poff == k).astype(jnp.float32) * lp[:, k * EMBED:(k + 1) * EMBED]
    x = (jnp.dot(uemb, w1u_ref[...], preferred_element_type=jnp.float32)
         + jnp.dot(pemb, w1p_ref[...], preferred_element_type=jnp.float32)
         + b1_ref[...])
    x = jnp.maximum(x, 0.0)
    o_ref[...] = jnp.sum(x * w2_ref[...], axis=1, keepdims=True) + b2_ref[...]


def _mlp(ids2, u_lines, p_lines, w1u, w1p, b1_2d, w2t, b2_2d):
    grid = (BATCH // _BLK,)
    return pl.pallas_call(
        _mlp_body,
        grid=grid,
        in_specs=[
            pl.BlockSpec((_BLK, 2), lambda i: (i, 0)),
            pl.BlockSpec((_BLK, LINE), lambda i: (i, 0)),
            pl.BlockSpec((_BLK, LINE), lambda i: (i, 0)),
            pl.BlockSpec((EMBED, HIDDEN), lambda i: (0, 0)),
            pl.BlockSpec((EMBED, HIDDEN), lambda i: (0, 0)),
            pl.BlockSpec((1, HIDDEN), lambda i: (0, 0)),
            pl.BlockSpec((1, HIDDEN), lambda i: (0, 0)),
            pl.BlockSpec((1, 1), lambda i: (0, 0)),
        ],
        out_specs=pl.BlockSpec((_BLK, 1), lambda i: (i, 0)),
        out_shape=jax.ShapeDtypeStruct((BATCH, 1), jnp.float32),
    )(ids2, u_lines, p_lines, w1u, w1p, b1_2d, w2t, b2_2d)


def kernel(user_ids, post_ids, user_table, post_table, W1, b1, W2, b2):
    u_pack = _pack(user_table.T)
    u_lines = _sc_gather(user_ids, u_pack)
    p_pack = _pack(post_table.T)
    p_lines = _sc_gather(post_ids, p_pack)
    ids2 = jnp.stack([user_ids, post_ids], axis=1)
    return _mlp(
        ids2, u_lines, p_lines,
        W1[:EMBED], W1[EMBED:],
        b1.reshape(1, HIDDEN),
        W2.reshape(1, HIDDEN),
        b2.reshape(1, 1),
    )
